# Initial kernel scaffold; baseline (speedup 1.0000x reference)
import jax
import jax.numpy as jnp
from jax.experimental import pallas as pl


def kernel(inputs, conv0_w, conv0_b, conv1_w, conv1_b, gcn_w0, gcn_b0, gcn_w1, gcn_b1, gcn_w2, gcn_b2, gcn_w3, gcn_b3, attn_w, attn_b, ln_g, ln_b, fc_w, fc_b):
    raise NotImplementedError("write your pallas kernel here")


# speedup vs baseline: 1.1890x; 1.1890x over previous
import functools

import jax
import jax.numpy as jnp
from jax import lax
from jax.experimental import pallas as pl
from jax.experimental.pallas import tpu as pltpu

B = 64
S = 2048
D = 128
K = 15
NC = 256
N = S // 4


def _shift_down(z, o, rows):
    return jnp.concatenate([jnp.zeros((o, z.shape[1]), z.dtype), z[: rows - o]], axis=0)


def _shift_up(z, o, rows):
    return jnp.concatenate([z[o:], jnp.zeros((o, z.shape[1]), z.dtype)], axis=0)


def _dot(a, b):
    return jnp.dot(a, b, preferred_element_type=jnp.float32)


def _gelu(x):
    return 0.5 * x * (1.0 + lax.erf(x * 0.7071067811865475))


def _fwd(x0_ref, x1_ref, x2_ref, x3_ref, c0w_ref, c0b_ref, c1w_ref, c1b_ref,
         gw0_ref, gb0_ref, gw1_ref, gb1_ref, gw2_ref, gb2_ref, gw3_ref, gb3_ref,
         aw_ref, ab_ref, lg_ref, lb_ref, fw_ref, fb_ref, o_ref):
    x0 = x0_ref[0]
    x1 = x1_ref[0]
    x2 = x2_ref[0]
    x3 = x3_ref[0]
    w0 = c0w_ref[0]
    w1 = c0w_ref[1]
    w2 = c0w_ref[2]
    b0 = c0b_ref[:]

    x3m = _shift_down(x3, 1, N)
    x0p = _shift_up(x0, 1, N)
    pe = 0.5 * (jnp.maximum(x3m * w0 + x0 * w1 + x1 * w2 + b0, 0.0)
                + jnp.maximum(x0 * w0 + x1 * w1 + x2 * w2 + b0, 0.0))
    po = 0.5 * (jnp.maximum(x1 * w0 + x2 * w1 + x3 * w2 + b0, 0.0)
                + jnp.maximum(x2 * w0 + x3 * w1 + x0p * w2 + b0, 0.0))

    b1 = c1b_ref[:]
    pom = _shift_down(po, 1, N)
    pep = _shift_up(pe, 1, N)
    ye = jnp.maximum(_dot(pom, c1w_ref[0]) + _dot(pe, c1w_ref[1])
                     + _dot(po, c1w_ref[2]) + b1, 0.0)
    yo = jnp.maximum(_dot(pe, c1w_ref[0]) + _dot(po, c1w_ref[1])
                     + _dot(pep, c1w_ref[2]) + b1, 0.0)
    xg = 0.5 * (ye + yo)

    ii = lax.broadcasted_iota(jnp.int32, (N, 1), 0)
    i = ii.astype(jnp.float32)
    deg = jnp.minimum(i, float(K)) + jnp.minimum(float(N - 1) - i, float(K)) + 1.0
    dinv = lax.rsqrt(deg)

    for w_ref, b_ref in ((gw0_ref, gb0_ref), (gw1_ref, gb1_ref),
                         (gw2_ref, gb2_ref), (gw3_ref, gb3_ref)):
        z = xg * dinv
        s = z
        for o in range(1, K + 1):
            s = s + _shift_down(z, o, N) + _shift_up(z, o, N)
        h = s * dinv
        h = _gelu(_dot(h, w_ref[:]) + b_ref[:])
        xg = xg + h

    scores = _dot(xg, aw_ref[:]) + ab_ref[0, 0]
    scores = scores - jnp.max(scores)
    e = jnp.exp(scores)
    a = e / jnp.sum(e)
    pooled = jnp.sum(a * xg, axis=0, keepdims=True)

    mu = jnp.mean(pooled, axis=-1, keepdims=True)
    var = jnp.mean((pooled - mu) ** 2, axis=-1, keepdims=True)
    pooled = (pooled - mu) * lax.rsqrt(var + 1e-6) * lg_ref[:] + lb_ref[:]

    o_ref[0] = _dot(pooled, fw_ref[:]) + fb_ref[:]


def kernel(inputs, conv0_w, conv0_b, conv1_w, conv1_b,
           gcn_w0, gcn_b0, gcn_w1, gcn_b1, gcn_w2, gcn_b2, gcn_w3, gcn_b3,
           attn_w, attn_b, ln_g, ln_b, fc_w, fc_b):
    xr = inputs.reshape(B, N, 4)
    xs0 = xr[:, :, 0:1]
    xs1 = xr[:, :, 1:2]
    xs2 = xr[:, :, 2:3]
    xs3 = xr[:, :, 3:4]
    c0b = conv0_b.reshape(1, D)
    c1b = conv1_b.reshape(1, D)
    gb0 = gcn_b0.reshape(1, D)
    gb1 = gcn_b1.reshape(1, D)
    gb2 = gcn_b2.reshape(1, D)
    gb3 = gcn_b3.reshape(1, D)
    ab = attn_b.reshape(1, 1)
    lg = ln_g.reshape(1, D)
    lb = ln_b.reshape(1, D)
    fb = fc_b.reshape(1, NC)

    def full(arr):
        nd = arr.ndim
        return pl.BlockSpec(arr.shape, lambda b: (0,) * nd)

    operands = (xs0, xs1, xs2, xs3, conv0_w, c0b, conv1_w, c1b,
                gcn_w0, gb0, gcn_w1, gb1, gcn_w2, gb2, gcn_w3, gb3,
                attn_w, ab, lg, lb, fc_w, fb)
    in_specs = [pl.BlockSpec((1, N, 1), lambda b: (b, 0, 0)) for _ in range(4)]
    in_specs += [full(a) for a in operands[4:]]

    out = pl.pallas_call(
        _fwd,
        grid=(B,),
        in_specs=in_specs,
        out_specs=pl.BlockSpec((1, 1, NC), lambda b: (b, 0, 0)),
        out_shape=jax.ShapeDtypeStruct((B, 1, NC), jnp.float32),
        compiler_params=pltpu.CompilerParams(
            dimension_semantics=("parallel",),
        ),
    )(*operands)
    return (out[:, 0, :],)


# speedup vs baseline: 1.4553x; 1.2240x over previous
import functools

import jax
import jax.numpy as jnp
from jax import lax
from jax.experimental import pallas as pl
from jax.experimental.pallas import tpu as pltpu

B = 64
S = 2048
D = 128
K = 15
NC = 256
N = S // 4


def _shift_down(z, o, rows):
    return jnp.concatenate([jnp.zeros((o, z.shape[1]), z.dtype), z[: rows - o]], axis=0)


def _shift_up(z, o, rows):
    return jnp.concatenate([z[o:], jnp.zeros((o, z.shape[1]), z.dtype)], axis=0)


def _dot(a, b):
    return jnp.dot(a, b, preferred_element_type=jnp.float32)


def _gelu(x):
    return 0.5 * x * (1.0 + lax.erf(x * 0.7071067811865475))


def _fwd(x4_ref, a6_ref, c0b_ref, c1w_ref, c1b_ref,
         gw0_ref, gb0_ref, gw1_ref, gb1_ref, gw2_ref, gb2_ref, gw3_ref, gb3_ref,
         aw_ref, lg_ref, lb_ref, fw_ref, fb_ref, o_ref):
    x4 = x4_ref[0]
    x3m = _shift_down(x4[:, 3:4], 1, N)
    x0p = _shift_up(x4[:, 0:1], 1, N)
    xc = jnp.concatenate([x4, x3m, x0p], axis=1)

    y4 = _dot(xc, a6_ref[:])
    b0 = c0b_ref[:]
    pe = 0.5 * (jnp.maximum(y4[:, :D] + b0, 0.0)
                + jnp.maximum(y4[:, D:2 * D] + b0, 0.0))
    po = 0.5 * (jnp.maximum(y4[:, 2 * D:3 * D] + b0, 0.0)
                + jnp.maximum(y4[:, 3 * D:] + b0, 0.0))

    b1 = c1b_ref[:]
    pom = _shift_down(po, 1, N)
    pep = _shift_up(pe, 1, N)
    ye = jnp.maximum(_dot(pom, c1w_ref[0]) + _dot(pe, c1w_ref[1])
                     + _dot(po, c1w_ref[2]) + b1, 0.0)
    yo = jnp.maximum(_dot(pe, c1w_ref[0]) + _dot(po, c1w_ref[1])
                     + _dot(pep, c1w_ref[2]) + b1, 0.0)
    xg = 0.5 * (ye + yo)

    ii = lax.broadcasted_iota(jnp.int32, (N, D), 0)
    i = ii.astype(jnp.float32)
    deg = jnp.minimum(i, float(K)) + jnp.minimum(float(N - 1) - i, float(K)) + 1.0
    dinv = lax.rsqrt(deg)

    for w_ref, b_ref in ((gw0_ref, gb0_ref), (gw1_ref, gb1_ref),
                         (gw2_ref, gb2_ref), (gw3_ref, gb3_ref)):
        z = xg * dinv
        u1 = z + _shift_up(z, 1, N)
        u2 = u1 + _shift_up(u1, 2, N)
        u3 = u2 + _shift_up(u2, 4, N)
        w15u = u3 + _shift_up(u2, 8, N) + _shift_up(u1, 12, N) + _shift_up(z, 14, N)
        d1 = z + _shift_down(z, 1, N)
        d2 = d1 + _shift_down(d1, 2, N)
        d3 = d2 + _shift_down(d2, 4, N)
        w15d = d3 + _shift_down(d2, 8, N) + _shift_down(d1, 12, N) + _shift_down(z, 14, N)
        s = z + _shift_up(w15u, 1, N) + _shift_down(w15d, 1, N)
        h = s * dinv
        h = _gelu(_dot(h, w_ref[:]) + b_ref[:])
        xg = xg + h

    sb = _dot(xg, aw_ref[:])
    sb = sb - jnp.max(sb)
    eb = jnp.exp(sb)
    se = jnp.sum(eb, axis=0, keepdims=True)
    pooled = jnp.sum(eb * xg, axis=0, keepdims=True) / se

    mu = jnp.mean(pooled, axis=-1, keepdims=True)
    var = jnp.mean((pooled - mu) ** 2, axis=-1, keepdims=True)
    pooled = (pooled - mu) * lax.rsqrt(var + 1e-6) * lg_ref[:] + lb_ref[:]

    o_ref[0] = _dot(pooled, fw_ref[:]) + fb_ref[:]


def kernel(inputs, conv0_w, conv0_b, conv1_w, conv1_b,
           gcn_w0, gcn_b0, gcn_w1, gcn_b1, gcn_w2, gcn_b2, gcn_w3, gcn_b3,
           attn_w, attn_b, ln_g, ln_b, fc_w, fc_b):
    xr = inputs.reshape(B, N, 4)
    w0, w1, w2 = conv0_w[0, 0], conv0_w[1, 0], conv0_w[2, 0]
    zD = jnp.zeros((D,), jnp.float32)
    a6 = jnp.stack([
        jnp.concatenate([w1, w0, zD, zD]),
        jnp.concatenate([w2, w1, w0, zD]),
        jnp.concatenate([zD, w2, w1, w0]),
        jnp.concatenate([zD, zD, w2, w1]),
        jnp.concatenate([w0, zD, zD, zD]),
        jnp.concatenate([zD, zD, zD, w2]),
    ], axis=0)
    aw_t = jnp.tile(attn_w, (1, D))
    c0b = conv0_b.reshape(1, D)
    c1b = conv1_b.reshape(1, D)
    gb0 = gcn_b0.reshape(1, D)
    gb1 = gcn_b1.reshape(1, D)
    gb2 = gcn_b2.reshape(1, D)
    gb3 = gcn_b3.reshape(1, D)
    del attn_b
    lg = ln_g.reshape(1, D)
    lb = ln_b.reshape(1, D)
    fb = fc_b.reshape(1, NC)

    def full(arr):
        nd = arr.ndim
        return pl.BlockSpec(arr.shape, lambda b: (0,) * nd)

    operands = (xr, a6, c0b, conv1_w, c1b,
                gcn_w0, gb0, gcn_w1, gb1, gcn_w2, gb2, gcn_w3, gb3,
                aw_t, lg, lb, fc_w, fb)
    in_specs = [pl.BlockSpec((1, N, 4), lambda b: (b, 0, 0))]
    in_specs += [full(a) for a in operands[1:]]

    out = pl.pallas_call(
        _fwd,
        grid=(B,),
        in_specs=in_specs,
        out_specs=pl.BlockSpec((1, 1, NC), lambda b: (b, 0, 0)),
        out_shape=jax.ShapeDtypeStruct((B, 1, NC), jnp.float32),
        compiler_params=pltpu.CompilerParams(
            dimension_semantics=("parallel",),
        ),
    )(*operands)
    return (out[:, 0, :],)


# speedup vs baseline: 1.8970x; 1.3034x over previous
import functools

import jax
import jax.numpy as jnp
from jax import lax
from jax.experimental import pallas as pl
from jax.experimental.pallas import tpu as pltpu

B = 64
S = 2048
D = 128
K = 15
NC = 256
N = S // 4


def _shift_down(z, o, rows):
    return jnp.concatenate([jnp.zeros((o, z.shape[1]), z.dtype), z[: rows - o]], axis=0)


def _shift_up(z, o, rows):
    return jnp.concatenate([z[o:], jnp.zeros((o, z.shape[1]), z.dtype)], axis=0)


def _dot(a, b):
    return jnp.dot(a, b, preferred_element_type=jnp.float32)


def _gelu(x):
    return 0.5 * x * (1.0 + lax.erf(x * 0.7071067811865475))


def _fwd(x4_ref, a6_ref, c0b_ref, c1w_ref, c1b_ref,
         gw0_ref, gb0_ref, gw1_ref, gb1_ref, gw2_ref, gb2_ref, gw3_ref, gb3_ref,
         aw_ref, lg_ref, lb_ref, fw_ref, fb_ref, o_ref):
    x4 = x4_ref[0]
    x3m = _shift_down(x4[:, 3:4], 1, N)
    x0p = _shift_up(x4[:, 0:1], 1, N)
    xc = jnp.concatenate([x4, x3m, x0p], axis=1)

    y4 = _dot(xc, a6_ref[:])
    b0 = c0b_ref[:]
    pe = jnp.maximum(y4[:, :D] + b0, 0.0) + jnp.maximum(y4[:, D:2 * D] + b0, 0.0)
    po = jnp.maximum(y4[:, 2 * D:3 * D] + b0, 0.0) + jnp.maximum(y4[:, 3 * D:] + b0, 0.0)

    b1 = c1b_ref[:]
    pom = _shift_down(po, 1, N)
    pep = _shift_up(pe, 1, N)
    ye = jnp.maximum(_dot(pom, c1w_ref[0]) + _dot(pe, c1w_ref[1])
                     + _dot(po, c1w_ref[2]) + b1, 0.0)
    yo = jnp.maximum(_dot(pe, c1w_ref[0]) + _dot(po, c1w_ref[1])
                     + _dot(pep, c1w_ref[2]) + b1, 0.0)
    xg = ye + yo

    ii = lax.broadcasted_iota(jnp.int32, (N, D), 0)
    i = ii.astype(jnp.float32)
    deg = jnp.minimum(i, float(K)) + jnp.minimum(float(N - 1) - i, float(K)) + 1.0
    dinv = lax.rsqrt(deg)

    for w_ref, b_ref in ((gw0_ref, gb0_ref), (gw1_ref, gb1_ref),
                         (gw2_ref, gb2_ref), (gw3_ref, gb3_ref)):
        z = xg * dinv
        p = z
        for o in (1, 2, 4, 8, 16, 32, 64, 128, 256):
            p = p + _shift_down(p, o, N)
        up = jnp.concatenate(
            [p[K:], jnp.broadcast_to(p[N - 1:N], (K, D))], axis=0)
        s = up - _shift_down(p, K + 1, N)
        h = s * dinv
        h = _gelu(_dot(h, w_ref[:]) + b_ref[:])
        xg = xg + h

    sb = _dot(xg, aw_ref[:])
    sb = sb - jnp.max(sb)
    eb = jnp.exp(sb)
    se = jnp.sum(eb, axis=0, keepdims=True)
    pooled = jnp.sum(eb * xg, axis=0, keepdims=True) / se

    mu = jnp.mean(pooled, axis=-1, keepdims=True)
    var = jnp.mean((pooled - mu) ** 2, axis=-1, keepdims=True)
    pooled = (pooled - mu) * lax.rsqrt(var + 1e-6) * lg_ref[:] + lb_ref[:]

    o_ref[0] = _dot(pooled, fw_ref[:]) + fb_ref[:]


def kernel(inputs, conv0_w, conv0_b, conv1_w, conv1_b,
           gcn_w0, gcn_b0, gcn_w1, gcn_b1, gcn_w2, gcn_b2, gcn_w3, gcn_b3,
           attn_w, attn_b, ln_g, ln_b, fc_w, fc_b):
    xr = inputs.reshape(B, N, 4)
    w0, w1, w2 = conv0_w[0, 0], conv0_w[1, 0], conv0_w[2, 0]
    zD = jnp.zeros((D,), jnp.float32)
    a6 = jnp.stack([
        jnp.concatenate([w1, w0, zD, zD]),
        jnp.concatenate([w2, w1, w0, zD]),
        jnp.concatenate([zD, w2, w1, w0]),
        jnp.concatenate([zD, zD, w2, w1]),
        jnp.concatenate([w0, zD, zD, zD]),
        jnp.concatenate([zD, zD, zD, w2]),
    ], axis=0)
    aw_t = jnp.tile(attn_w, (1, D))
    c0b = conv0_b.reshape(1, D)
    c1w = conv1_w * 0.25
    c1b = conv1_b.reshape(1, D) * 0.5
    gb0 = gcn_b0.reshape(1, D)
    gb1 = gcn_b1.reshape(1, D)
    gb2 = gcn_b2.reshape(1, D)
    gb3 = gcn_b3.reshape(1, D)
    del attn_b
    lg = ln_g.reshape(1, D)
    lb = ln_b.reshape(1, D)
    fb = fc_b.reshape(1, NC)

    def full(arr):
        nd = arr.ndim
        return pl.BlockSpec(arr.shape, lambda b: (0,) * nd)

    operands = (xr, a6, c0b, c1w, c1b,
                gcn_w0, gb0, gcn_w1, gb1, gcn_w2, gb2, gcn_w3, gb3,
                aw_t, lg, lb, fc_w, fb)
    in_specs = [pl.BlockSpec((1, N, 4), lambda b: (b, 0, 0))]
    in_specs += [full(a) for a in operands[1:]]

    out = pl.pallas_call(
        _fwd,
        grid=(B,),
        in_specs=in_specs,
        out_specs=pl.BlockSpec((1, 1, NC), lambda b: (b, 0, 0)),
        out_shape=jax.ShapeDtypeStruct((B, 1, NC), jnp.float32),
        compiler_params=pltpu.CompilerParams(
            dimension_semantics=("parallel",),
        ),
    )(*operands)
    return (out[:, 0, :],)


# speedup vs baseline: 1.9631x; 1.0349x over previous
import functools

import jax
import jax.numpy as jnp
import numpy as np
from jax import lax
from jax.experimental import pallas as pl
from jax.experimental.pallas import tpu as pltpu

B = 64
S = 2048
D = 128
K = 15
NC = 256
N = S // 4


def _shift_down(z, o, rows):
    return jnp.concatenate([jnp.zeros((o, z.shape[1]), z.dtype), z[: rows - o]], axis=0)


def _shift_up(z, o, rows):
    return jnp.concatenate([z[o:], jnp.zeros((o, z.shape[1]), z.dtype)], axis=0)


def _dot(a, b):
    return jnp.dot(a, b, preferred_element_type=jnp.float32)


def _gelu(x):
    return 0.5 * x * (1.0 + lax.erf(x * 0.7071067811865475))


def _fwd(x4_ref, a6_ref, c0b_ref, c1w_ref, c1b_ref,
         gw0_ref, gb0_ref, gw1_ref, gb1_ref, gw2_ref, gb2_ref, gw3_ref, gb3_ref,
         aw_ref, lg_ref, lb_ref, fw_ref, fb_ref, o_ref):
    x4 = x4_ref[0]
    x3m = _shift_down(x4[:, 3:4], 1, N)
    x0p = _shift_up(x4[:, 0:1], 1, N)
    xc = jnp.concatenate([x4, x3m, x0p], axis=1)

    b0 = c0b_ref[:]
    pe = (jnp.maximum(_dot(xc, a6_ref[:, :D]) + b0, 0.0)
          + jnp.maximum(_dot(xc, a6_ref[:, D:2 * D]) + b0, 0.0))
    po = (jnp.maximum(_dot(xc, a6_ref[:, 2 * D:3 * D]) + b0, 0.0)
          + jnp.maximum(_dot(xc, a6_ref[:, 3 * D:]) + b0, 0.0))

    b1 = c1b_ref[:]
    pom = _shift_down(po, 1, N)
    pep = _shift_up(pe, 1, N)
    ye = jnp.maximum(_dot(pom, c1w_ref[0]) + _dot(pe, c1w_ref[1])
                     + _dot(po, c1w_ref[2]) + b1, 0.0)
    yo = jnp.maximum(_dot(pe, c1w_ref[0]) + _dot(po, c1w_ref[1])
                     + _dot(pep, c1w_ref[2]) + b1, 0.0)
    xg = ye + yo

    E = 16
    cK = float(1.0 / np.sqrt(2 * K + 1))
    ii = lax.broadcasted_iota(jnp.int32, (E, D), 0).astype(jnp.float32)
    etop = lax.rsqrt(jnp.minimum(ii, float(K)) + float(K) + 1.0)
    ebot = lax.rsqrt(jnp.minimum(float(N - 1) - (float(N - E) + ii), float(K))
                     + float(K) + 1.0)

    def _dscale(v):
        return jnp.concatenate(
            [v[:E] * etop, v[E:N - E] * cK, v[N - E:] * ebot], axis=0)

    for w_ref, b_ref in ((gw0_ref, gb0_ref), (gw1_ref, gb1_ref),
                         (gw2_ref, gb2_ref), (gw3_ref, gb3_ref)):
        z = _dscale(xg)
        p = z
        for o in (1, 2, 4, 8, 16, 32, 64, 128, 256):
            p = p + _shift_down(p, o, N)
        up = jnp.concatenate(
            [p[K:], jnp.broadcast_to(p[N - 1:N], (K, D))], axis=0)
        s = up - _shift_down(p, K + 1, N)
        h = _dscale(s)
        h = _gelu(_dot(h, w_ref[:]) + b_ref[:])
        xg = xg + h

    sb = _dot(xg, aw_ref[:])
    sb = sb - jnp.max(sb)
    eb = jnp.exp(sb)
    se = jnp.sum(eb, axis=0, keepdims=True)
    pooled = jnp.sum(eb * xg, axis=0, keepdims=True) / se

    mu = jnp.mean(pooled, axis=-1, keepdims=True)
    var = jnp.mean((pooled - mu) ** 2, axis=-1, keepdims=True)
    pooled = (pooled - mu) * lax.rsqrt(var + 1e-6) * lg_ref[:] + lb_ref[:]

    o_ref[0] = _dot(pooled, fw_ref[:]) + fb_ref[:]


def kernel(inputs, conv0_w, conv0_b, conv1_w, conv1_b,
           gcn_w0, gcn_b0, gcn_w1, gcn_b1, gcn_w2, gcn_b2, gcn_w3, gcn_b3,
           attn_w, attn_b, ln_g, ln_b, fc_w, fc_b):
    xr = inputs.reshape(B, N, 4)
    w0, w1, w2 = conv0_w[0, 0], conv0_w[1, 0], conv0_w[2, 0]
    zD = jnp.zeros((D,), jnp.float32)
    a6 = jnp.stack([
        jnp.concatenate([w1, w0, zD, zD]),
        jnp.concatenate([w2, w1, w0, zD]),
        jnp.concatenate([zD, w2, w1, w0]),
        jnp.concatenate([zD, zD, w2, w1]),
        jnp.concatenate([w0, zD, zD, zD]),
        jnp.concatenate([zD, zD, zD, w2]),
    ], axis=0)
    aw_t = jnp.tile(attn_w, (1, D))
    c0b = conv0_b.reshape(1, D)
    c1w = conv1_w * 0.25
    c1b = conv1_b.reshape(1, D) * 0.5
    gb0 = gcn_b0.reshape(1, D)
    gb1 = gcn_b1.reshape(1, D)
    gb2 = gcn_b2.reshape(1, D)
    gb3 = gcn_b3.reshape(1, D)
    del attn_b
    lg = ln_g.reshape(1, D)
    lb = ln_b.reshape(1, D)
    fb = fc_b.reshape(1, NC)

    def full(arr):
        nd = arr.ndim
        return pl.BlockSpec(arr.shape, lambda b: (0,) * nd)

    operands = (xr, a6, c0b, c1w, c1b,
                gcn_w0, gb0, gcn_w1, gb1, gcn_w2, gb2, gcn_w3, gb3,
                aw_t, lg, lb, fc_w, fb)
    in_specs = [pl.BlockSpec((1, N, 4), lambda b: (b, 0, 0))]
    in_specs += [full(a) for a in operands[1:]]

    out = pl.pallas_call(
        _fwd,
        grid=(B,),
        in_specs=in_specs,
        out_specs=pl.BlockSpec((1, 1, NC), lambda b: (b, 0, 0)),
        out_shape=jax.ShapeDtypeStruct((B, 1, NC), jnp.float32),
        compiler_params=pltpu.CompilerParams(
            dimension_semantics=("parallel",),
        ),
    )(*operands)
    return (out[:, 0, :],)
